# baseline (speedup 1.0000x reference)
import functools

import jax
import jax.numpy as jnp
from jax import lax
from jax.experimental import pallas as pl
from jax.experimental.pallas import tpu as pltpu
from jax.experimental.pallas import tpu_sc as plsc

_NC = 2
_NS = 16
_NW = _NC * _NS
_CHUNK = 128
_SEL_BLK = 512


def _sc_gather(W2, idx2, n, D2):
    b_per_w = n // _NW
    n_chunks = b_per_w // _CHUNK
    mesh = plsc.VectorSubcoreMesh(core_axis_name="c", subcore_axis_name="s")

    @functools.partial(
        pl.kernel,
        mesh=mesh,
        out_type=jax.ShapeDtypeStruct((n, D2), W2.dtype),
        scratch_types=[
            pltpu.VMEM((b_per_w,), jnp.int32),
            pltpu.VMEM((_CHUNK, D2), jnp.float32),
            pltpu.SemaphoreType.DMA,
        ],
    )
    def gather_kernel(w_hbm, idx_hbm, out_hbm, idx_v, rows_v, sem):
        wid = lax.axis_index("s") * _NC + lax.axis_index("c")
        base = wid * b_per_w
        pltpu.sync_copy(idx_hbm.at[pl.ds(base, b_per_w)], idx_v)

        @pl.loop(0, n_chunks)
        def _(i):
            off = i * _CHUNK
            pltpu.async_copy(
                w_hbm.at[idx_v.at[pl.ds(off, _CHUNK)]], rows_v, sem
            ).wait()
            pltpu.sync_copy(rows_v, out_hbm.at[pl.ds(base + off, _CHUNK)])

    return gather_kernel(W2, idx2)


def _select_kernel(rows_ref, idx_ref, out_ref):
    parity = (idx_ref[0].reshape(_SEL_BLK, 1) & 1) == 1
    rows = rows_ref[...]
    out_ref[...] = jnp.where(parity, rows[:, 64:], rows[:, :64])


def _tc_select(rows, idx, n, D):
    nb = n // _SEL_BLK
    idx3 = idx.reshape(nb, 1, _SEL_BLK)
    return pl.pallas_call(
        _select_kernel,
        grid=(nb,),
        in_specs=[
            pl.BlockSpec((_SEL_BLK, 2 * D), lambda i: (i, 0)),
            pl.BlockSpec((1, 1, _SEL_BLK), lambda i: (i, 0, 0)),
        ],
        out_specs=pl.BlockSpec((_SEL_BLK, D), lambda i: (i, 0)),
        out_shape=jax.ShapeDtypeStruct((n, D), rows.dtype),
    )(rows, idx3)


def kernel(x, W):
    B, L = x.shape
    V, D = W.shape
    n = B * L
    idx = x.reshape(n)
    W2 = W.reshape(V // 2, 2 * D)
    rows = _sc_gather(W2, idx >> 1, n, 2 * D)
    out = _tc_select(rows, idx, n, D)
    return out.reshape(B, L, D)


# speedup vs baseline: 1.5557x; 1.5557x over previous
import functools

import jax
import jax.numpy as jnp
from jax import lax
from jax.experimental import pallas as pl
from jax.experimental.pallas import tpu as pltpu
from jax.experimental.pallas import tpu_sc as plsc

_NC = 2
_NS = 16
_NW = _NC * _NS
_CHUNK = 128
_SEL_BLK = 4096


def _sc_gather(W2, idx2, n):
    b_per_w = n // _NW
    n_chunks = b_per_w // _CHUNK
    mesh = plsc.VectorSubcoreMesh(core_axis_name="c", subcore_axis_name="s")

    nslot = 4
    assert n_chunks % nslot == 0

    @functools.partial(
        pl.kernel,
        mesh=mesh,
        out_type=jax.ShapeDtypeStruct((n, 128), W2.dtype),
        scratch_types=[
            pltpu.VMEM((b_per_w,), jnp.int32),
            pltpu.VMEM((nslot, _CHUNK, 128), jnp.float32),
            pltpu.SemaphoreType.DMA((nslot,)),
            pltpu.SemaphoreType.DMA((nslot,)),
        ],
    )
    def gather_kernel(w_hbm, idx_hbm, out_hbm, idx_v, rows_v, gsem, wsem):
        wid = lax.axis_index("s") * _NC + lax.axis_index("c")
        base = wid * b_per_w
        pltpu.sync_copy(idx_hbm.at[pl.ds(base, b_per_w)], idx_v)

        def gather_desc(i, slot):
            return pltpu.make_async_copy(
                w_hbm.at[idx_v.at[pl.ds(i * _CHUNK, _CHUNK)]],
                rows_v.at[slot],
                gsem.at[slot],
            )

        def write_desc(i, slot):
            return pltpu.make_async_copy(
                rows_v.at[slot],
                out_hbm.at[pl.ds(base + i * _CHUNK, _CHUNK)],
                wsem.at[slot],
            )

        for s in range(nslot):
            gather_desc(s, s).start()

        @pl.loop(0, n_chunks // nslot)
        def _(i4):
            i = i4 * nslot
            for s in range(nslot):
                gather_desc(i + s, s).wait()
                write_desc(i + s, s).start()
            @pl.when(i + nslot < n_chunks)
            def _():
                for s in range(nslot):
                    write_desc(i + s, s).wait()
                    gather_desc(i + nslot + s, s).start()

        for s in range(nslot):
            write_desc(n_chunks - nslot + s, s).wait()

    return gather_kernel(W2, idx2)


def _select_kernel(rows_ref, idx_ref, out_ref):
    parity = (idx_ref[0, 0].reshape(_SEL_BLK, 1) & 1) == 1
    rows = rows_ref[...]
    out_ref[...] = jnp.where(parity, rows[:, 64:], rows[:, :64])


def _tc_select(rows, idx, n, D):
    nb = n // _SEL_BLK
    idx3 = idx.reshape(nb, 1, _SEL_BLK)
    return pl.pallas_call(
        _select_kernel,
        grid=(nb,),
        in_specs=[
            pl.BlockSpec((_SEL_BLK, 128), lambda i: (i, 0)),
            pl.BlockSpec((1, 1, _SEL_BLK), lambda i: (i, 0, 0)),
        ],
        out_specs=pl.BlockSpec((_SEL_BLK, D), lambda i: (i, 0)),
        out_shape=jax.ShapeDtypeStruct((n, D), rows.dtype),
        compiler_params=pltpu.CompilerParams(
            dimension_semantics=("parallel",),
        ),
    )(rows, idx3)


def kernel(x, W):
    B, L = x.shape
    V, D = W.shape
    n = B * L
    idx = x.reshape(n)
    W2 = W.reshape(V // 2, 2 * D)
    rows = _sc_gather(W2, idx >> 1, n)
    out = _tc_select(rows, idx, n, D)
    return out.reshape(B, L, D)
